# baseline (speedup 1.0000x reference)
import functools

import jax
import jax.numpy as jnp
from jax import lax
from jax.experimental import pallas as pl
from jax.experimental.pallas import tpu as pltpu
from jax.experimental.pallas import tpu_sc as plsc

NX, NY = 1000, 1000
DIM = 64
XMIN, XMAX, YMIN, YMAX = 0.0, 1.0, 0.0, 1.0
DX = (XMAX - XMIN) / NX
DY = (YMAX - YMIN) / NY

L = 16
CB = 128


@functools.lru_cache(maxsize=None)
def _build(n_points: int):
    info = plsc.get_sparse_core_info()
    nc, ns = info.num_cores, info.num_subcores
    nw = nc * ns
    npw = n_points // nw
    nu = npw // CB
    assert npw * nw == n_points and nu * CB == npw and nu % 2 == 0

    mesh = plsc.VectorSubcoreMesh(core_axis_name="c", subcore_axis_name="s")

    @functools.partial(
        pl.kernel,
        out_type=jax.ShapeDtypeStruct((n_points, DIM), jnp.float32),
        mesh=mesh,
        scratch_types=[
            [pltpu.VMEM((CB,), jnp.float32) for _ in range(2)],
            [pltpu.VMEM((CB,), jnp.float32) for _ in range(2)],
            [pltpu.VMEM((CB,), jnp.int32) for _ in range(2)],
            [pltpu.VMEM((CB,), jnp.int32) for _ in range(2)],
            [pltpu.VMEM((CB, 2 * DIM), jnp.float32) for _ in range(2)],
            [pltpu.VMEM((CB, 2 * DIM), jnp.float32) for _ in range(2)],
            [pltpu.VMEM((CB, DIM), jnp.float32) for _ in range(2)],
            [pltpu.SemaphoreType.DMA for _ in range(2)],
            [pltpu.SemaphoreType.DMA for _ in range(2)],
            [pltpu.SemaphoreType.DMA for _ in range(2)],
        ],
    )
    def lookup(xs_hbm, ys_hbm, comb_hbm, out_hbm,
               cxv, cyv, idxx, idxy, bufx, bufy, outb, semc, semg, semo):
        wid = lax.axis_index("s") * nc + lax.axis_index("c")
        wbase = wid * npw

        def fire_coords(u, cs):
            gb = wbase + u * CB
            pltpu.async_copy(xs_hbm.at[pl.ds(gb, CB)], cxv[cs], semc[cs])
            pltpu.async_copy(ys_hbm.at[pl.ds(gb, CB)], cyv[cs], semc[cs])

        def wait_coords(cs):
            pltpu.make_async_copy(
                xs_hbm.at[pl.ds(0, CB)], cxv[cs], semc[cs]).wait()
            pltpu.make_async_copy(
                ys_hbm.at[pl.ds(0, CB)], cyv[cs], semc[cs]).wait()

        def front(s):
            for j in range(CB // L):
                c = pl.ds(j * L, L)
                x = cxv[s][c]
                y = cyv[s][c]
                xi = ((x - XMIN) / DX).astype(jnp.int32)
                yi = ((y - YMIN) / DY).astype(jnp.int32)
                xi = jnp.where((xi > NX) | (xi < 0), NX, xi)
                yi = jnp.where((yi > NY) | (yi < 0), NY, yi)
                idxx[s][c] = xi
                idxy[s][c] = yi
            pltpu.async_copy(comb_hbm.at[idxx[s]], bufx[s], semg[s])
            pltpu.async_copy(comb_hbm.at[idxy[s]], bufy[s], semg[s])

        def back(u, s):
            @pl.when(u >= 2)
            def _():
                pltpu.make_async_copy(
                    outb[s], out_hbm.at[pl.ds(0, CB)], semo[s]).wait()
            pltpu.make_async_copy(
                comb_hbm.at[idxx[s]], bufx[s], semg[s]).wait()
            pltpu.make_async_copy(
                comb_hbm.at[idxy[s]], bufy[s], semg[s]).wait()

            def add_row(i, c):
                for col in range(DIM // L):
                    sa = pl.ds(col * L, L)
                    sb = pl.ds(DIM + col * L, L)
                    outb[s][i, sa] = bufx[s][i, sa] + bufy[s][i, sb]
                return c
            lax.fori_loop(0, CB, add_row, 0)
            pltpu.async_copy(
                outb[s], out_hbm.at[pl.ds(wbase + u * CB, CB)], semo[s])

        fire_coords(0, 0)

        def pair_body(q, carry):
            u0 = 2 * q
            u1 = u0 + 1
            wait_coords(0)
            fire_coords(u1, 1)
            front(0)

            @pl.when(q > 0)
            def _():
                back(u0 - 1, 1)

            wait_coords(1)

            @pl.when(u0 + 2 < nu)
            def _():
                fire_coords(u0 + 2, 0)
            front(1)
            back(u0, 0)
            return carry

        lax.fori_loop(0, nu // 2, pair_body, 0)
        back(nu - 1, 1)
        for s in range(2):
            pltpu.make_async_copy(
                outb[s], out_hbm.at[pl.ds(0, CB)], semo[s]).wait()

    return lookup


def kernel(T, ebdx_w, ebdy_w):
    b, h, _ = T.shape
    n = b * h
    xs = jnp.minimum(T[:, :, 0], 2.0).reshape(n)
    ys = jnp.minimum(T[:, :, 1], 2.0).reshape(n)
    comb = jnp.concatenate([ebdx_w, ebdy_w], axis=1)
    out = _build(n)(xs, ys, comb)
    return out.reshape(b, h, DIM)


# speedup vs baseline: 1.0051x; 1.0051x over previous
import functools

import jax
import jax.numpy as jnp
from jax import lax
from jax.experimental import pallas as pl
from jax.experimental.pallas import tpu as pltpu
from jax.experimental.pallas import tpu_sc as plsc

NX, NY = 1000, 1000
DIM = 64
XMIN, XMAX, YMIN, YMAX = 0.0, 1.0, 0.0, 1.0
DX = (XMAX - XMIN) / NX
DY = (YMAX - YMIN) / NY

L = 16
CB = 80


@functools.lru_cache(maxsize=None)
def _build(n_points: int):
    info = plsc.get_sparse_core_info()
    nc, ns = info.num_cores, info.num_subcores
    nw = nc * ns
    npw = n_points // nw
    nu = npw // CB
    assert npw * nw == n_points and nu * CB == npw and nu % 3 == 2

    mesh = plsc.VectorSubcoreMesh(core_axis_name="c", subcore_axis_name="s")

    @functools.partial(
        pl.kernel,
        out_type=jax.ShapeDtypeStruct((n_points, DIM), jnp.float32),
        mesh=mesh,
        scratch_types=[
            [pltpu.VMEM((CB,), jnp.float32) for _ in range(3)],
            [pltpu.VMEM((CB,), jnp.float32) for _ in range(3)],
            [pltpu.VMEM((CB,), jnp.int32) for _ in range(3)],
            [pltpu.VMEM((CB,), jnp.int32) for _ in range(3)],
            [pltpu.VMEM((CB, 2 * DIM), jnp.float32) for _ in range(3)],
            [pltpu.VMEM((CB, 2 * DIM), jnp.float32) for _ in range(3)],
            [pltpu.VMEM((CB, DIM), jnp.float32) for _ in range(3)],
            [pltpu.SemaphoreType.DMA for _ in range(3)],
            [pltpu.SemaphoreType.DMA for _ in range(3)],
            [pltpu.SemaphoreType.DMA for _ in range(3)],
        ],
    )
    def lookup(xs_hbm, ys_hbm, comb_hbm, out_hbm,
               cxv, cyv, idxx, idxy, bufx, bufy, outb, semc, semg, semo):
        wid = lax.axis_index("s") * nc + lax.axis_index("c")
        wbase = wid * npw

        def fire_coords(u, cs):
            gb = wbase + u * CB
            pltpu.async_copy(xs_hbm.at[pl.ds(gb, CB)], cxv[cs], semc[cs])
            pltpu.async_copy(ys_hbm.at[pl.ds(gb, CB)], cyv[cs], semc[cs])

        def wait_coords(cs):
            pltpu.make_async_copy(
                xs_hbm.at[pl.ds(0, CB)], cxv[cs], semc[cs]).wait()
            pltpu.make_async_copy(
                ys_hbm.at[pl.ds(0, CB)], cyv[cs], semc[cs]).wait()

        def stage(u, s):
            for j in range(CB // L):
                c = pl.ds(j * L, L)
                x = cxv[s][c]
                y = cyv[s][c]
                xi = ((x - XMIN) / DX).astype(jnp.int32)
                yi = ((y - YMIN) / DY).astype(jnp.int32)
                xi = jnp.where((xi > NX) | (xi < 0), NX, xi)
                yi = jnp.where((yi > NY) | (yi < 0), NY, yi)
                idxx[s][c] = xi
                idxy[s][c] = yi
            pltpu.async_copy(comb_hbm.at[idxx[s]], bufx[s], semg[s])
            pltpu.async_copy(comb_hbm.at[idxy[s]], bufy[s], semg[s])

            @pl.when(u + 3 < nu)
            def _():
                fire_coords(u + 3, s)

        def drain(u, s):
            @pl.when(u >= 3)
            def _():
                pltpu.make_async_copy(
                    outb[s], out_hbm.at[pl.ds(0, CB)], semo[s]).wait()
            pltpu.make_async_copy(
                comb_hbm.at[idxx[s]], bufx[s], semg[s]).wait()
            pltpu.make_async_copy(
                comb_hbm.at[idxy[s]], bufy[s], semg[s]).wait()

            def add_row(i, c):
                for col in range(DIM // L):
                    sa = pl.ds(col * L, L)
                    sb = pl.ds(DIM + col * L, L)
                    outb[s][i, sa] = bufx[s][i, sa] + bufy[s][i, sb]
                return c
            lax.fori_loop(0, CB, add_row, 0)
            pltpu.async_copy(
                outb[s], out_hbm.at[pl.ds(wbase + u * CB, CB)], semo[s])

        for s in range(3):
            fire_coords(s, s)
        wait_coords(0)
        stage(0, 0)
        wait_coords(1)
        stage(1, 1)

        def trip_body(q, carry):
            b0 = 3 * q
            for j in range(3):
                drain(b0 + j, j)
                wait_coords((j + 2) % 3)
                stage(b0 + j + 2, (j + 2) % 3)
            return carry

        lax.fori_loop(0, nu // 3, trip_body, 0)
        drain(nu - 2, (nu - 2) % 3)
        drain(nu - 1, (nu - 1) % 3)
        for s in range(3):
            pltpu.make_async_copy(
                outb[s], out_hbm.at[pl.ds(0, CB)], semo[s]).wait()

    return lookup


def kernel(T, ebdx_w, ebdy_w):
    b, h, _ = T.shape
    n = b * h
    xs = jnp.minimum(T[:, :, 0], 2.0).reshape(n)
    ys = jnp.minimum(T[:, :, 1], 2.0).reshape(n)
    comb = jnp.concatenate([ebdx_w, ebdy_w], axis=1)
    out = _build(n)(xs, ys, comb)
    return out.reshape(b, h, DIM)


# speedup vs baseline: 1.6298x; 1.6215x over previous
import functools

import jax
import jax.numpy as jnp
from jax import lax
from jax.experimental import pallas as pl
from jax.experimental.pallas import tpu as pltpu
from jax.experimental.pallas import tpu_sc as plsc

NX, NY = 1000, 1000
DIM = 64
XMIN, XMAX, YMIN, YMAX = 0.0, 1.0, 0.0, 1.0
DX = (XMAX - XMIN) / NX
DY = (YMAX - YMIN) / NY

L = 16
CB = 80


@functools.lru_cache(maxsize=None)
def _build(n_points: int):
    info = plsc.get_sparse_core_info()
    nc, ns = info.num_cores, info.num_subcores
    nw = nc * ns
    npw = n_points // nw
    nu = npw // CB
    assert npw * nw == n_points and nu * CB == npw and nu % 3 == 2

    mesh = plsc.VectorSubcoreMesh(core_axis_name="c", subcore_axis_name="s")

    @functools.partial(
        pl.kernel,
        out_type=jax.ShapeDtypeStruct((n_points, DIM), jnp.float32),
        mesh=mesh,
        scratch_types=[
            [pltpu.VMEM((CB,), jnp.float32) for _ in range(3)],
            [pltpu.VMEM((CB,), jnp.float32) for _ in range(3)],
            [pltpu.VMEM((CB,), jnp.int32) for _ in range(3)],
            [pltpu.VMEM((CB,), jnp.int32) for _ in range(3)],
            [pltpu.VMEM((CB, 2 * DIM), jnp.float32) for _ in range(3)],
            [pltpu.VMEM((CB, 2 * DIM), jnp.float32) for _ in range(3)],
            [pltpu.VMEM((CB, DIM), jnp.float32) for _ in range(3)],
            [pltpu.SemaphoreType.DMA for _ in range(3)],
            [pltpu.SemaphoreType.DMA for _ in range(3)],
            [pltpu.SemaphoreType.DMA for _ in range(3)],
            pltpu.VMEM_SHARED((NX + 1, 2 * DIM), jnp.float32),
        ],
    )
    def lookup(xs_hbm, ys_hbm, comb_hbm, out_hbm,
               cxv, cyv, idxx, idxy, bufx, bufy, outb, semc, semg, semo,
               comb_sh):
        wid = lax.axis_index("s") * nc + lax.axis_index("c")
        wbase = wid * npw

        @pl.when(lax.axis_index("s") == 0)
        def _():
            pltpu.sync_copy(comb_hbm, comb_sh)
        plsc.subcore_barrier()

        def fire_coords(u, cs):
            gb = wbase + u * CB
            pltpu.async_copy(xs_hbm.at[pl.ds(gb, CB)], cxv[cs], semc[cs])
            pltpu.async_copy(ys_hbm.at[pl.ds(gb, CB)], cyv[cs], semc[cs])

        def wait_coords(cs):
            pltpu.make_async_copy(
                xs_hbm.at[pl.ds(0, CB)], cxv[cs], semc[cs]).wait()
            pltpu.make_async_copy(
                ys_hbm.at[pl.ds(0, CB)], cyv[cs], semc[cs]).wait()

        def stage(u, s):
            for j in range(CB // L):
                c = pl.ds(j * L, L)
                x = cxv[s][c]
                y = cyv[s][c]
                xi = ((x - XMIN) / DX).astype(jnp.int32)
                yi = ((y - YMIN) / DY).astype(jnp.int32)
                xi = jnp.where((xi > NX) | (xi < 0), NX, xi)
                yi = jnp.where((yi > NY) | (yi < 0), NY, yi)
                idxx[s][c] = xi
                idxy[s][c] = yi
            pltpu.async_copy(comb_sh.at[idxx[s]], bufx[s], semg[s])
            pltpu.async_copy(comb_sh.at[idxy[s]], bufy[s], semg[s])

            @pl.when(u + 3 < nu)
            def _():
                fire_coords(u + 3, s)

        def drain(u, s):
            @pl.when(u >= 3)
            def _():
                pltpu.make_async_copy(
                    outb[s], out_hbm.at[pl.ds(0, CB)], semo[s]).wait()
            pltpu.make_async_copy(
                comb_sh.at[idxx[s]], bufx[s], semg[s]).wait()
            pltpu.make_async_copy(
                comb_sh.at[idxy[s]], bufy[s], semg[s]).wait()

            def add_row(i, c):
                for col in range(DIM // L):
                    sa = pl.ds(col * L, L)
                    sb = pl.ds(DIM + col * L, L)
                    outb[s][i, sa] = bufx[s][i, sa] + bufy[s][i, sb]
                return c
            lax.fori_loop(0, CB, add_row, 0)
            pltpu.async_copy(
                outb[s], out_hbm.at[pl.ds(wbase + u * CB, CB)], semo[s])

        for s in range(3):
            fire_coords(s, s)
        wait_coords(0)
        stage(0, 0)
        wait_coords(1)
        stage(1, 1)

        def trip_body(q, carry):
            b0 = 3 * q
            for j in range(3):
                drain(b0 + j, j)
                wait_coords((j + 2) % 3)
                stage(b0 + j + 2, (j + 2) % 3)
            return carry

        lax.fori_loop(0, nu // 3, trip_body, 0)
        drain(nu - 2, (nu - 2) % 3)
        drain(nu - 1, (nu - 1) % 3)
        for s in range(3):
            pltpu.make_async_copy(
                outb[s], out_hbm.at[pl.ds(0, CB)], semo[s]).wait()

    return lookup


def kernel(T, ebdx_w, ebdy_w):
    b, h, _ = T.shape
    n = b * h
    xs = jnp.minimum(T[:, :, 0], 2.0).reshape(n)
    ys = jnp.minimum(T[:, :, 1], 2.0).reshape(n)
    comb = jnp.concatenate([ebdx_w, ebdy_w], axis=1)
    out = _build(n)(xs, ys, comb)
    return out.reshape(b, h, DIM)


# speedup vs baseline: 1.6673x; 1.0230x over previous
import functools

import jax
import jax.numpy as jnp
from jax import lax
from jax.experimental import pallas as pl
from jax.experimental.pallas import tpu as pltpu
from jax.experimental.pallas import tpu_sc as plsc

NX, NY = 1000, 1000
DIM = 64
XMIN, XMAX, YMIN, YMAX = 0.0, 1.0, 0.0, 1.0
DX = (XMAX - XMIN) / NX
DY = (YMAX - YMIN) / NY

L = 16
CB = 128


@functools.lru_cache(maxsize=None)
def _build(n_points: int):
    info = plsc.get_sparse_core_info()
    nc, ns = info.num_cores, info.num_subcores
    nw = nc * ns
    npw = n_points // nw
    nu = npw // CB
    assert npw * nw == n_points and nu * CB == npw and nu % 2 == 0

    mesh = plsc.VectorSubcoreMesh(core_axis_name="c", subcore_axis_name="s")

    @functools.partial(
        pl.kernel,
        out_type=jax.ShapeDtypeStruct((n_points, DIM), jnp.float32),
        mesh=mesh,
        scratch_types=[
            [pltpu.VMEM((CB,), jnp.float32) for _ in range(2)],
            [pltpu.VMEM((CB,), jnp.float32) for _ in range(2)],
            [pltpu.VMEM((CB,), jnp.int32) for _ in range(2)],
            [pltpu.VMEM((CB,), jnp.int32) for _ in range(2)],
            [pltpu.VMEM((CB, 2 * DIM), jnp.float32) for _ in range(2)],
            [pltpu.VMEM((CB, 2 * DIM), jnp.float32) for _ in range(2)],
            [pltpu.VMEM((CB, DIM), jnp.float32) for _ in range(2)],
            [pltpu.SemaphoreType.DMA for _ in range(2)],
            [pltpu.SemaphoreType.DMA for _ in range(2)],
            [pltpu.SemaphoreType.DMA for _ in range(2)],
            pltpu.VMEM_SHARED((NX + 1, 2 * DIM), jnp.float32),
        ],
    )
    def lookup(xs_hbm, ys_hbm, comb_hbm, out_hbm,
               cxv, cyv, idxx, idxy, bufx, bufy, outb, semc, semg, semo,
               comb_sh):
        wid = lax.axis_index("s") * nc + lax.axis_index("c")
        wbase = wid * npw

        @pl.when(lax.axis_index("s") == 0)
        def _():
            pltpu.sync_copy(comb_hbm, comb_sh)
        plsc.subcore_barrier()

        def fire_coords(u, cs):
            gb = wbase + u * CB
            pltpu.async_copy(xs_hbm.at[pl.ds(gb, CB)], cxv[cs], semc[cs])
            pltpu.async_copy(ys_hbm.at[pl.ds(gb, CB)], cyv[cs], semc[cs])

        def wait_coords(cs):
            pltpu.make_async_copy(
                xs_hbm.at[pl.ds(0, CB)], cxv[cs], semc[cs]).wait()
            pltpu.make_async_copy(
                ys_hbm.at[pl.ds(0, CB)], cyv[cs], semc[cs]).wait()

        def stage(u, s):
            for j in range(CB // L):
                c = pl.ds(j * L, L)
                x = cxv[s][c]
                y = cyv[s][c]
                xi = ((x - XMIN) / DX).astype(jnp.int32)
                yi = ((y - YMIN) / DY).astype(jnp.int32)
                xi = jnp.where((xi > NX) | (xi < 0), NX, xi)
                yi = jnp.where((yi > NY) | (yi < 0), NY, yi)
                idxx[s][c] = xi
                idxy[s][c] = yi
            pltpu.async_copy(comb_sh.at[idxx[s]], bufx[s], semg[s])
            pltpu.async_copy(comb_sh.at[idxy[s]], bufy[s], semg[s])

            @pl.when(u + 2 < nu)
            def _():
                fire_coords(u + 2, s)

        def drain(u, s):
            @pl.when(u >= 2)
            def _():
                pltpu.make_async_copy(
                    outb[s], out_hbm.at[pl.ds(0, CB)], semo[s]).wait()
            pltpu.make_async_copy(
                comb_sh.at[idxx[s]], bufx[s], semg[s]).wait()
            pltpu.make_async_copy(
                comb_sh.at[idxy[s]], bufy[s], semg[s]).wait()

            def add_rows(i, c):
                for dr in range(2):
                    r = 2 * i + dr
                    for col in range(DIM // L):
                        sa = pl.ds(col * L, L)
                        sb = pl.ds(DIM + col * L, L)
                        outb[s][r, sa] = bufx[s][r, sa] + bufy[s][r, sb]
                return c
            lax.fori_loop(0, CB // 2, add_rows, 0)
            pltpu.async_copy(
                outb[s], out_hbm.at[pl.ds(wbase + u * CB, CB)], semo[s])

        for s in range(2):
            fire_coords(s, s)
        wait_coords(0)
        stage(0, 0)

        def pair_body(q, carry):
            b0 = 2 * q
            wait_coords(1)
            stage(b0 + 1, 1)
            drain(b0, 0)

            @pl.when(b0 + 2 < nu)
            def _():
                wait_coords(0)
                stage(b0 + 2, 0)
            drain(b0 + 1, 1)
            return carry

        lax.fori_loop(0, nu // 2, pair_body, 0)
        for s in range(2):
            pltpu.make_async_copy(
                outb[s], out_hbm.at[pl.ds(0, CB)], semo[s]).wait()

    return lookup


def kernel(T, ebdx_w, ebdy_w):
    b, h, _ = T.shape
    n = b * h
    xs = jnp.minimum(T[:, :, 0], 2.0).reshape(n)
    ys = jnp.minimum(T[:, :, 1], 2.0).reshape(n)
    comb = jnp.concatenate([ebdx_w, ebdy_w], axis=1)
    out = _build(n)(xs, ys, comb)
    return out.reshape(b, h, DIM)


# speedup vs baseline: 1.6827x; 1.0092x over previous
import functools

import jax
import jax.numpy as jnp
from jax import lax
from jax.experimental import pallas as pl
from jax.experimental.pallas import tpu as pltpu
from jax.experimental.pallas import tpu_sc as plsc

NX, NY = 1000, 1000
DIM = 64
XMIN, XMAX, YMIN, YMAX = 0.0, 1.0, 0.0, 1.0
DX = (XMAX - XMIN) / NX
DY = (YMAX - YMIN) / NY

L = 16
CB = 128


@functools.lru_cache(maxsize=None)
def _build(n_points: int):
    info = plsc.get_sparse_core_info()
    nc, ns = info.num_cores, info.num_subcores
    nw = nc * ns
    npw = n_points // nw
    nu = npw // CB
    assert npw * nw == n_points and nu * CB == npw and nu % 2 == 0

    mesh = plsc.VectorSubcoreMesh(core_axis_name="c", subcore_axis_name="s")

    @functools.partial(
        pl.kernel,
        out_type=jax.ShapeDtypeStruct((n_points, DIM), jnp.float32),
        mesh=mesh,
        scratch_types=[
            [pltpu.VMEM((CB,), jnp.float32) for _ in range(2)],
            [pltpu.VMEM((CB,), jnp.float32) for _ in range(2)],
            [pltpu.VMEM((CB,), jnp.int32) for _ in range(2)],
            [pltpu.VMEM((CB,), jnp.int32) for _ in range(2)],
            [pltpu.VMEM((CB, 2 * DIM), jnp.float32) for _ in range(2)],
            [pltpu.VMEM((CB, 2 * DIM), jnp.float32) for _ in range(2)],
            [pltpu.VMEM((CB, DIM), jnp.float32) for _ in range(2)],
            [pltpu.SemaphoreType.DMA for _ in range(2)],
            [pltpu.SemaphoreType.DMA for _ in range(2)],
            [pltpu.SemaphoreType.DMA for _ in range(2)],
            pltpu.VMEM_SHARED((NX + 1, 2 * DIM), jnp.float32),
        ],
    )
    def lookup(xs_hbm, ys_hbm, comb_hbm, out_hbm,
               cxv, cyv, idxx, idxy, bufx, bufy, outb, semc, semg, semo,
               comb_sh):
        wid = lax.axis_index("s") * nc + lax.axis_index("c")
        wbase = wid * npw

        @pl.when(lax.axis_index("s") == 0)
        def _():
            pltpu.sync_copy(comb_hbm, comb_sh)
        plsc.subcore_barrier()

        def fire_coords(u, cs):
            gb = wbase + u * CB
            pltpu.async_copy(xs_hbm.at[pl.ds(gb, CB)], cxv[cs], semc[cs])
            pltpu.async_copy(ys_hbm.at[pl.ds(gb, CB)], cyv[cs], semc[cs])

        def wait_coords(cs):
            pltpu.make_async_copy(
                xs_hbm.at[pl.ds(0, CB)], cxv[cs], semc[cs]).wait()
            pltpu.make_async_copy(
                ys_hbm.at[pl.ds(0, CB)], cyv[cs], semc[cs]).wait()

        def stage(u, s):
            for j in range(CB // L):
                c = pl.ds(j * L, L)
                x = cxv[s][c]
                y = cyv[s][c]
                xi = ((x - XMIN) / DX).astype(jnp.int32)
                yi = ((y - YMIN) / DY).astype(jnp.int32)
                xi = jnp.where((xi > NX) | (xi < 0), NX, xi)
                yi = jnp.where((yi > NY) | (yi < 0), NY, yi)
                idxx[s][c] = xi
                idxy[s][c] = yi
            pltpu.async_copy(comb_sh.at[idxx[s]], bufx[s], semg[s])
            pltpu.async_copy(comb_sh.at[idxy[s]], bufy[s], semg[s])

            @pl.when(u + 2 < nu)
            def _():
                fire_coords(u + 2, s)

        def drain(u, s):
            @pl.when(u >= 2)
            def _():
                pltpu.make_async_copy(
                    outb[s], out_hbm.at[pl.ds(0, CB)], semo[s]).wait()
            pltpu.make_async_copy(
                comb_sh.at[idxx[s]], bufx[s], semg[s]).wait()
            pltpu.make_async_copy(
                comb_sh.at[idxy[s]], bufy[s], semg[s]).wait()

            def add_rows(i, c):
                for dr in range(2):
                    r = 2 * i + dr
                    for col in range(DIM // L):
                        sa = pl.ds(col * L, L)
                        sb = pl.ds(DIM + col * L, L)
                        outb[s][r, sa] = bufx[s][r, sa] + bufy[s][r, sb]
                return c
            lax.fori_loop(0, CB // 2, add_rows, 0)
            pltpu.async_copy(
                outb[s], out_hbm.at[pl.ds(wbase + u * CB, CB)], semo[s])

        for s in range(2):
            fire_coords(s, s)
        wait_coords(0)
        stage(0, 0)

        def pair_body(q, carry):
            b0 = 2 * q
            wait_coords(1)
            stage(b0 + 1, 1)
            drain(b0, 0)

            @pl.when(b0 + 2 < nu)
            def _():
                wait_coords(0)
                stage(b0 + 2, 0)
            drain(b0 + 1, 1)
            return carry

        lax.fori_loop(0, nu // 2, pair_body, 0)
        for s in range(2):
            pltpu.make_async_copy(
                outb[s], out_hbm.at[pl.ds(0, CB)], semo[s]).wait()

    return lookup


def kernel(T, ebdx_w, ebdy_w):
    b, h, _ = T.shape
    n = b * h
    e0 = jnp.array([1.0, 0.0], dtype=jnp.float32)
    e1 = jnp.array([0.0, 1.0], dtype=jnp.float32)
    xs = jnp.einsum("blc,c->bl", T, e0).reshape(n)
    ys = jnp.einsum("blc,c->bl", T, e1).reshape(n)
    comb = jnp.concatenate([ebdx_w, ebdy_w], axis=1)
    out = _build(n)(xs, ys, comb)
    return out.reshape(b, h, DIM)
